# baseline (speedup 1.0000x reference)
import functools

import jax
import jax.numpy as jnp
from jax.experimental import pallas as pl
from jax.experimental.pallas import tpu as pltpu

_H, _W = 128, 128


def _scatter_body(idx_ref, xt_ref, out_ref, *, M: int, K2: int, nsub: int):
    idx = idx_ref[0, 0, :]
    for j2 in range(nsub):
        cols = jax.lax.broadcasted_iota(jnp.int32, (M, K2), 1) + j2 * K2
        onehot = (idx[:, None] == cols).astype(jnp.float32)
        out_ref[0, :, j2 * K2:(j2 + 1) * K2] = jax.lax.dot_general(
            xt_ref[0], onehot, (((1,), (0,)), ((), ())),
            preferred_element_type=jnp.float32)


def kernel(x, spatial_size, location):
    B, M, N = x.shape
    H, W = _H, _W
    HW = H * W
    index = (location[:, :, 0] * spatial_size[1] + location[:, :, 1]) % HW
    index = index.reshape(B, 1, M)
    xt = jnp.transpose(x, (0, 2, 1))

    K2 = 1024
    nsub = HW // K2
    out = pl.pallas_call(
        functools.partial(_scatter_body, M=M, K2=K2, nsub=nsub),
        grid=(B,),
        in_specs=[
            pl.BlockSpec((1, 1, M), lambda b: (b, 0, 0)),
            pl.BlockSpec((1, N, M), lambda b: (b, 0, 0)),
        ],
        out_specs=pl.BlockSpec((1, N, HW), lambda b: (b, 0, 0)),
        out_shape=jax.ShapeDtypeStruct((B, N, HW), jnp.float32),
        compiler_params=pltpu.CompilerParams(
            dimension_semantics=("parallel",)),
    )(index, xt)
    return out.reshape(B, N, H, W)
